# baseline (speedup 1.0000x reference)
import functools

import jax
import jax.numpy as jnp
from jax import lax
from jax.experimental import pallas as pl
from jax.experimental.pallas import tpu as pltpu
from jax.experimental.pallas import tpu_sc as plsc

_L = 16


def _tc_depad(table, rows_blk=8192):
    n, d = table.shape
    grid = (n + rows_blk - 1) // rows_blk
    out_rows = grid * (rows_blk // 8)

    def body(t_ref, o_ref):
        t3 = t_ref[...].reshape(rows_blk // 8, 8, d)
        o_ref[...] = jnp.concatenate([t3[:, k, :] for k in range(8)], axis=1)

    return pl.pallas_call(
        body,
        grid=(grid,),
        in_specs=[pl.BlockSpec((rows_blk, d), lambda i: (i, 0))],
        out_specs=pl.BlockSpec((rows_blk // 8, 128), lambda i: (i, 0)),
        out_shape=jax.ShapeDtypeStruct((out_rows, 128), jnp.float32),
    )(table)


def _sc_gather_t(table128, idx_raw, off_pattern, bsz, fc, d):
    info = plsc.get_sparse_core_info()
    nc, ns = info.num_cores, info.num_subcores
    nw = nc * ns
    b_per_w = bsz // nw
    sub = _L * fc
    n_blk = b_per_w // 128

    mesh = plsc.VectorSubcoreMesh(core_axis_name="c", subcore_axis_name="s")

    @functools.partial(
        pl.kernel,
        mesh=mesh,
        out_type=jax.ShapeDtypeStruct((fc * d, bsz), jnp.float32),
        scratch_types=[
            pltpu.VMEM((sub,), jnp.int32),
            pltpu.VMEM((sub,), jnp.int32),
            pltpu.VMEM((sub,), jnp.int32),
            pltpu.VMEM((sub, 128), jnp.float32),
            pltpu.VMEM((fc * d, 128), jnp.float32),
            pltpu.SemaphoreType.DMA,
        ],
        compiler_params=pltpu.CompilerParams(
            use_tc_tiling_on_sc=False, needs_layout_passes=False
        ),
    )
    def k(tab_hbm, idx_hbm, offp_hbm, out_hbm, g_v, r_v, off_v, grp_v, t_v,
          sem):
        wid = lax.axis_index("s") * nc + lax.axis_index("c")
        b0w = wid * b_per_w
        pltpu.sync_copy(offp_hbm, off_v)
        lanes = lax.iota(jnp.int32, _L)
        for blk in range(n_blk):
            b0 = b0w + blk * 128
            for sc_i in range(8):
                s0 = (b0 + sc_i * _L) * fc
                pltpu.sync_copy(idx_hbm.at[pl.ds(s0, sub)], g_v)
                for i in range(fc):
                    s = pl.ds(i * _L, _L)
                    full = g_v[s] + off_v[s]
                    g_v[s] = lax.shift_right_logical(full, 3)
                    r_v[s] = lax.bitwise_and(full, 7)
                pltpu.async_copy(tab_hbm.at[g_v], grp_v, sem).wait()

                def per_fg(f, carry):
                    j16 = (lanes * fc) + f
                    rr = plsc.load_gather(r_v, [j16])

                    def per_d(dd, c2):
                        val = plsc.load_gather(grp_v, [j16, rr * _L + dd])
                        t_v[f * _L + dd, pl.ds(sc_i * _L, _L)] = val
                        return c2

                    lax.fori_loop(0, d, per_d, 0)
                    return carry

                lax.fori_loop(0, fc, per_fg, 0)
            for f in range(fc):
                pltpu.sync_copy(
                    t_v.at[pl.ds(f * _L, _L), :],
                    out_hbm.at[pl.ds(f * _L, _L), pl.ds(b0, 128)],
                )

    return k(table128, idx_raw, off_pattern)


def _tc_assemble_t(x_t, w, b, cat_t, block_b=1024):
    f, bsz = x_t.shape
    d = w.shape[1]
    fcd = cat_t.shape[0]

    def body(x_ref, w_ref, b_ref, cat_ref, o_ref):
        o_ref[:f, :, :] = (
            x_ref[...][:, None, :] * w_ref[...][:, :, None]
            + b_ref[...][:, :, None]
        )
        o_ref[f:, :, :] = cat_ref[...].reshape(fcd // d, d, block_b)

    return pl.pallas_call(
        body,
        grid=(bsz // block_b,),
        in_specs=[
            pl.BlockSpec((f, block_b), lambda i: (0, i)),
            pl.BlockSpec((f, d), lambda i: (0, 0)),
            pl.BlockSpec((f, d), lambda i: (0, 0)),
            pl.BlockSpec((fcd, block_b), lambda i: (0, i)),
        ],
        out_specs=pl.BlockSpec((f + fcd // d, d, block_b), lambda i: (0, 0, i)),
        out_shape=jax.ShapeDtypeStruct((f + fcd // d, d, bsz), jnp.float32),
    )(x_t, w, b, cat_t)


def kernel(x_num, x_cat, num_weight, num_bias, cat_table, category_offsets):
    bsz, fc = x_cat.shape
    d = cat_table.shape[1]
    idx_raw = x_cat.astype(jnp.int32).reshape(-1)
    off_pattern = jnp.tile(category_offsets.astype(jnp.int32), _L)
    table128 = _tc_depad(cat_table)
    cat_t = _sc_gather_t(table128, idx_raw, off_pattern, bsz, fc, d)
    out_t = _tc_assemble_t(x_num.T, num_weight, num_bias, cat_t)
    return out_t.transpose(2, 0, 1)
